# baseline (speedup 1.0000x reference)
import functools

import jax
import jax.numpy as jnp
from jax import lax
from jax.experimental import pallas as pl
from jax.experimental.pallas import tpu as pltpu
from jax.experimental.pallas import tpu_sc as plsc

_SEGMENT_LENGTH = 512
_NUM_SEGMENTS = 32
_LANE_TILE = 128
_WB_CHUNK = 8


def _segment_starts(n: int) -> list[int]:
    l, s = _SEGMENT_LENGTH, _NUM_SEGMENTS
    starts = []
    for i in range(s):
        st = i * (n - l) // max(1, s - 1)
        if st + l > n:
            st = n - l
        starts.append(st)
    return starts


@jax.jit
def kernel(points):
    b_dim, n, c = points.shape
    l, s = _SEGMENT_LENGTH, _NUM_SEGMENTS
    starts = _segment_starts(n)
    buf_w = l + _LANE_TILE

    n_al = (n // _LANE_TILE) * _LANE_TILE
    tail_w = n - n_al

    spans = []
    for st in starts:
        a0 = (st // _LANE_TILE) * _LANE_TILE
        off = st - a0
        end = a0 + (buf_w if off else l)
        tail = max(0, min(end, st + l) - n_al)
        w = min(end, n_al) - a0
        spans.append((a0, off, w, tail))

    info = plsc.get_sparse_core_info()
    nc, ns = info.num_cores, info.num_subcores
    nw = nc * ns
    b_per_w = b_dim // nw

    mesh = plsc.VectorSubcoreMesh(core_axis_name="c", subcore_axis_name="s")

    @functools.partial(
        pl.kernel,
        mesh=mesh,
        out_type=jax.ShapeDtypeStruct((b_dim, s, c, l), points.dtype),
        scratch_types=[
            pltpu.VMEM((s, c, buf_w), points.dtype),
            pltpu.VMEM((s, c, buf_w), points.dtype),
            pltpu.VMEM((b_per_w, c, tail_w), points.dtype),
            pltpu.SemaphoreType.DMA,
            pltpu.SemaphoreType.DMA,
            pltpu.SemaphoreType.DMA,
        ],
        compiler_params=pltpu.CompilerParams(
            use_tc_tiling_on_sc=True, needs_layout_passes=False
        ),
    )
    def seg_sampler(
        points_hbm, tail_hbm, out_hbm, buf0, buf1, tbuf, sem0, sem1, wsem
    ):
        wid = lax.axis_index("s") * nc + lax.axis_index("c")
        base = wid * b_per_w
        bufs = (buf0, buf1)
        sems = (sem0, sem1)

        tp = pltpu.make_async_copy(
            tail_hbm.at[pl.ds(base, b_per_w)], tbuf, wsem
        )
        tp.start()

        def gather_copy(db, si):
            a0, off, w, tail = spans[si]
            return pltpu.make_async_copy(
                points_hbm.at[base + db, :, pl.ds(a0, w)],
                bufs[db].at[si, :, pl.ds(0, w)],
                sems[db],
            )

        for si in range(s):
            for db in range(b_per_w):
                gather_copy(db, si).start()
        tp.wait()

        lane = lax.iota(jnp.int32, 16)
        zero_v = jnp.zeros((16,), jnp.int32)
        one_v = jnp.ones((16,), jnp.int32)

        wbs = []
        for si, (a0, off, w, tail) in enumerate(spans):
            for db in range(b_per_w):
                gather_copy(db, si).wait()
            if off:
                main = l - tail
                idx_base = lane + off
                si_v = jnp.full((16,), si, jnp.int32)

                def shift_body(k, _, si_v=si_v, idx_base=idx_base):
                    idx = idx_base + k * 16
                    a0v = plsc.load_gather(buf0, [si_v, zero_v, idx])
                    a1v = plsc.load_gather(buf0, [si_v, one_v, idx])
                    b0v = plsc.load_gather(buf1, [si_v, zero_v, idx])
                    b1v = plsc.load_gather(buf1, [si_v, one_v, idx])
                    ksl = pl.ds(k * 16, 16)
                    buf0[si, 0, ksl] = a0v
                    buf0[si, 1, ksl] = a1v
                    buf1[si, 0, ksl] = b0v
                    buf1[si, 1, ksl] = b1v
                    return _

                lax.fori_loop(0, main // 16, shift_body, None, unroll=8)
                for j in range(main, l, 16):
                    jsl = pl.ds(j, 16)
                    tsl = pl.ds(j - main, 16)
                    buf0[si, 0, jsl] = tbuf[0, 0, tsl]
                    buf0[si, 1, jsl] = tbuf[0, 1, tsl]
                    buf1[si, 0, jsl] = tbuf[1, 0, tsl]
                    buf1[si, 1, jsl] = tbuf[1, 1, tsl]
            if si % _WB_CHUNK == _WB_CHUNK - 1:
                s0 = si - (_WB_CHUNK - 1)
                for db in range(b_per_w):
                    wb = pltpu.make_async_copy(
                        bufs[db].at[pl.ds(s0, _WB_CHUNK), :, pl.ds(0, l)],
                        out_hbm.at[base + db, pl.ds(s0, _WB_CHUNK)],
                        wsem,
                    )
                    wb.start()
                    wbs.append(wb)
        for wb in wbs:
            wb.wait()

    tail_in = points[:, n_al:, :].transpose(0, 2, 1)
    out = seg_sampler(points.transpose(0, 2, 1), tail_in)
    return out.transpose(0, 1, 3, 2)


# speedup vs baseline: 1.3085x; 1.3085x over previous
import functools

import jax
import jax.numpy as jnp
from jax import lax
from jax.experimental import pallas as pl
from jax.experimental.pallas import tpu as pltpu
from jax.experimental.pallas import tpu_sc as plsc

_SEGMENT_LENGTH = 512
_NUM_SEGMENTS = 32
_LANE_TILE = 128
_WB_CHUNK = 8


def _segment_starts(n: int) -> list[int]:
    l, s = _SEGMENT_LENGTH, _NUM_SEGMENTS
    starts = []
    for i in range(s):
        st = i * (n - l) // max(1, s - 1)
        if st + l > n:
            st = n - l
        starts.append(st)
    return starts


@jax.jit
def kernel(points):
    b_dim, n, c = points.shape
    l, s = _SEGMENT_LENGTH, _NUM_SEGMENTS
    starts = _segment_starts(n)
    buf_w = l + _LANE_TILE

    n_al = (n // _LANE_TILE) * _LANE_TILE
    tail_w = n - n_al

    spans = []
    for st in starts:
        a0 = (st // _LANE_TILE) * _LANE_TILE
        off = st - a0
        end = a0 + (buf_w if off else l)
        tail = max(0, min(end, st + l) - n_al)
        w = min(end, n_al) - a0
        spans.append((a0, off, w, tail))

    info = plsc.get_sparse_core_info()
    nc, ns = info.num_cores, info.num_subcores
    nw = nc * ns
    b_per_w = b_dim // nw

    mesh = plsc.VectorSubcoreMesh(core_axis_name="c", subcore_axis_name="s")

    @functools.partial(
        pl.kernel,
        mesh=mesh,
        out_type=jax.ShapeDtypeStruct((b_dim, s, c, l), points.dtype),
        scratch_types=[
            pltpu.VMEM((s, c, buf_w), points.dtype),
            pltpu.VMEM((s, c, buf_w), points.dtype),
            pltpu.VMEM((b_per_w, _WB_CHUNK, c, l), points.dtype),
            pltpu.VMEM((b_per_w, _WB_CHUNK, c, l), points.dtype),
            pltpu.VMEM((b_per_w, c, tail_w), points.dtype),
            pltpu.SemaphoreType.DMA,
            pltpu.SemaphoreType.DMA,
            pltpu.SemaphoreType.DMA,
        ],
        compiler_params=pltpu.CompilerParams(
            use_tc_tiling_on_sc=True, needs_layout_passes=False
        ),
    )
    def seg_sampler(
        points_hbm, tail_hbm, out_hbm, buf0, buf1, ob0, ob1, tbuf, sem0, sem1, wsem
    ):
        wid = lax.axis_index("s") * nc + lax.axis_index("c")
        base = wid * b_per_w
        bufs = (buf0, buf1)
        sems = (sem0, sem1)

        tp = pltpu.make_async_copy(
            tail_hbm.at[pl.ds(base, b_per_w)], tbuf, wsem
        )
        tp.start()

        def gather_copy(db, si):
            a0, off, w, tail = spans[si]
            return pltpu.make_async_copy(
                points_hbm.at[base + db, :, pl.ds(a0, w)],
                bufs[db].at[si, :, pl.ds(0, w)],
                sems[db],
            )

        for si in range(s):
            for db in range(b_per_w):
                gather_copy(db, si).start()
        tp.wait()

        lane = lax.iota(jnp.int32, 16)
        zero_v = jnp.zeros((16,), jnp.int32)
        one_v = jnp.ones((16,), jnp.int32)

        obs = (ob0, ob1)
        wbs = [None, None]
        final_wbs = []
        for si, (a0, off, w, tail) in enumerate(spans):
            chunk = si // _WB_CHUNK
            ob = obs[chunk % 2]
            ci = si % _WB_CHUNK
            if ci == 0 and wbs[chunk % 2] is not None:
                for wb in wbs[chunk % 2]:
                    wb.wait()
            for db in range(b_per_w):
                gather_copy(db, si).wait()
            main = l - tail
            idx_base = lane + off
            si_v = jnp.full((16,), si, jnp.int32)
            ci_v = jnp.full((16,), ci, jnp.int32)

            @plsc.parallel_loop(0, main // 16, unroll=4)
            def shift_body(k, si_v=si_v, ci=ci, idx_base=idx_base, ob=ob):
                idx = idx_base + k * 16
                a0v = plsc.load_gather(buf0, [si_v, zero_v, idx])
                a1v = plsc.load_gather(buf0, [si_v, one_v, idx])
                b0v = plsc.load_gather(buf1, [si_v, zero_v, idx])
                b1v = plsc.load_gather(buf1, [si_v, one_v, idx])
                ksl = pl.ds(k * 16, 16)
                ob[0, ci, 0, ksl] = a0v
                ob[0, ci, 1, ksl] = a1v
                ob[1, ci, 0, ksl] = b0v
                ob[1, ci, 1, ksl] = b1v

            for j in range(main, l, 16):
                jsl = pl.ds(j, 16)
                tsl = pl.ds(j - main, 16)
                ob[0, ci, 0, jsl] = tbuf[0, 0, tsl]
                ob[0, ci, 1, jsl] = tbuf[0, 1, tsl]
                ob[1, ci, 0, jsl] = tbuf[1, 0, tsl]
                ob[1, ci, 1, jsl] = tbuf[1, 1, tsl]
            if ci == _WB_CHUNK - 1:
                s0 = si - (_WB_CHUNK - 1)
                cur = []
                for db in range(b_per_w):
                    wb = pltpu.make_async_copy(
                        ob.at[db],
                        out_hbm.at[base + db, pl.ds(s0, _WB_CHUNK)],
                        wsem,
                    )
                    wb.start()
                    cur.append(wb)
                wbs[chunk % 2] = cur
                final_wbs.append(cur)
        for cur in final_wbs[-2:]:
            for wb in cur:
                wb.wait()

    tail_in = points[:, n_al:, :].transpose(0, 2, 1)
    out = seg_sampler(points.transpose(0, 2, 1), tail_in)
    return out.transpose(0, 1, 3, 2)
